# baseline (speedup 1.0000x reference)
import functools

import jax
import jax.numpy as jnp
from jax.experimental import pallas as pl

_NPOINT = 1024
_RADIUS = 0.4
_NSAMPLE = 32
_TOPK = 512
_FG = 512


def _fps_kernel(pts_ref, idx_ref, *, npoint):
    M = pts_ref.shape[1]
    x = pts_ref[0:1, :]
    y = pts_ref[1:2, :]
    z = pts_ref[2:3, :]
    lane = jax.lax.broadcasted_iota(jnp.int32, (1, M), 1)
    out_lane = jax.lax.broadcasted_iota(jnp.int32, (1, npoint), 1)

    def body(i, carry):
        dists, far, idxs = carry
        idxs = jnp.where(out_lane == i, far, idxs)
        sel = (lane == far).astype(jnp.float32)
        cx = jnp.sum(x * sel)
        cy = jnp.sum(y * sel)
        cz = jnp.sum(z * sel)
        d = (x - cx) ** 2 + (y - cy) ** 2 + (z - cz) ** 2
        dists = jnp.minimum(dists, d)
        mx = jnp.max(dists)
        far = jnp.min(jnp.where(dists == mx, lane, M)).astype(jnp.int32)
        return dists, far, idxs

    init = (
        jnp.full((1, M), 1e10, dtype=jnp.float32),
        jnp.int32(0),
        jnp.zeros((1, npoint), dtype=jnp.int32),
    )
    _, _, idxs = jax.lax.fori_loop(0, npoint, body, init)
    idx_ref[...] = idxs


def _fps_call(pts, npoint):
    B, _, M = pts.shape
    out = pl.pallas_call(
        functools.partial(_fps_kernel, npoint=npoint),
        grid=(B,),
        in_specs=[pl.BlockSpec((None, 3, M), lambda b: (b, 0, 0))],
        out_specs=pl.BlockSpec((None, 1, npoint), lambda b: (b, 0, 0)),
        out_shape=jax.ShapeDtypeStruct((B, 1, npoint), jnp.int32),
    )(pts)
    return out[:, 0, :]


def _mlp_kernel(x_ref, w0_ref, b0_ref, w1_ref, b1_ref, w2_ref, b2_ref,
                o_ref, *, ts, ns):
    x = x_ref[...]
    y = jnp.maximum(jnp.dot(x, w0_ref[...],
                            preferred_element_type=jnp.float32) + b0_ref[...], 0.0)
    y = jnp.maximum(jnp.dot(y, w1_ref[...],
                            preferred_element_type=jnp.float32) + b1_ref[...], 0.0)
    y = jnp.maximum(jnp.dot(y, w2_ref[...],
                            preferred_element_type=jnp.float32) + b2_ref[...], 0.0)
    o_ref[...] = jnp.max(y.reshape(ts, ns, y.shape[-1]), axis=1)


def _mlp_call(grouped, w0t, b0, w1t, b1, w2t, b2):
    B, SN, Cin = grouped.shape
    ns = _NSAMPLE
    S = SN // ns
    Cout = w2t.shape[1]
    ts = 128
    kern = functools.partial(_mlp_kernel, ts=ts, ns=ns)
    rep = lambda b, s: (0, 0)
    out = pl.pallas_call(
        kern,
        grid=(B, S // ts),
        in_specs=[
            pl.BlockSpec((None, ts * ns, Cin), lambda b, s: (b, s, 0)),
            pl.BlockSpec(w0t.shape, rep),
            pl.BlockSpec(b0.shape, rep),
            pl.BlockSpec(w1t.shape, rep),
            pl.BlockSpec(b1.shape, rep),
            pl.BlockSpec(w2t.shape, rep),
            pl.BlockSpec(b2.shape, rep),
        ],
        out_specs=pl.BlockSpec((None, ts, Cout), lambda b, s: (b, s, 0)),
        out_shape=jax.ShapeDtypeStruct((B, S, Cout), jnp.float32),
    )(grouped, w0t, b0, w1t, b1, w2t, b2)
    return out


def kernel(xyz, features, fbs_w, fbs_b, conv_w0, bn_g0, bn_b0,
           conv_w1, bn_g1, bn_b1, conv_w2, bn_g2, bn_b2):
    B, N, _ = xyz.shape

    scores = jnp.einsum('oc,bcn->bon', fbs_w, features) + fbs_b[None, :, None]
    soft = jax.nn.softmax(scores, axis=1)
    margin = soft[:, 1, :] - soft[:, 0, :]
    _, top_idx = jax.lax.top_k(margin, _TOPK)
    mask = jnp.zeros((B, N), dtype=jnp.int32).at[
        jnp.arange(B)[:, None], top_idx].set(1)
    pos_idx = jnp.sort(top_idx, axis=1).astype(jnp.int32)
    neg_idx = jnp.argsort(mask, axis=1).astype(jnp.int32)[:, : N - _TOPK]

    gather_rows = jax.vmap(lambda a, i: a[i])
    pos_xyz = gather_rows(xyz, pos_idx)
    neg_xyz = gather_rows(xyz, neg_idx)
    sel_pos = _fps_call(jnp.transpose(pos_xyz, (0, 2, 1)), _FG)
    sel_neg = _fps_call(jnp.transpose(neg_xyz, (0, 2, 1)), _NPOINT - _FG)
    new_pos = jnp.take_along_axis(pos_idx, sel_pos, axis=1)
    new_neg = jnp.take_along_axis(neg_idx, sel_neg, axis=1)
    indices = jnp.concatenate([new_pos, new_neg], axis=1)
    new_xyz = gather_rows(xyz, indices)

    def ball_one(xyz_b, new_xyz_b):
        sqr = jnp.sum((new_xyz_b[:, None, :] - xyz_b[None, :, :]) ** 2, axis=-1)
        gidx = jnp.broadcast_to(jnp.arange(N, dtype=jnp.int32), sqr.shape)
        gidx = jnp.where(sqr > _RADIUS ** 2, N, gidx)
        neg_top, _ = jax.lax.top_k(-gidx, _NSAMPLE)
        gidx = -neg_top
        first = gidx[:, :1]
        return jnp.where(gidx == N, first, gidx)

    gidx = jax.vmap(ball_one)(xyz, new_xyz)

    feat_t = jnp.transpose(features, (0, 2, 1))
    gx = gather_rows(xyz, gidx.reshape(B, -1)).reshape(B, _NPOINT, _NSAMPLE, 3)
    gx = gx - new_xyz[:, :, None, :]
    gf = gather_rows(feat_t, gidx.reshape(B, -1)).reshape(
        B, _NPOINT, _NSAMPLE, feat_t.shape[-1])
    grouped = jnp.concatenate([gx, gf], axis=-1)
    grouped = grouped.reshape(B, _NPOINT * _NSAMPLE, -1)

    w0t = (conv_w0 * bn_g0[:, None]).T
    w1t = (conv_w1 * bn_g1[:, None]).T
    w2t = (conv_w2 * bn_g2[:, None]).T
    out = _mlp_call(grouped, w0t, bn_b0[None, :], w1t, bn_b1[None, :],
                    w2t, bn_b2[None, :])
    new_features = jnp.transpose(out, (0, 2, 1))
    return new_xyz, new_features


# speedup vs baseline: 1.1455x; 1.1455x over previous
import functools

import jax
import jax.numpy as jnp
from jax.experimental import pallas as pl

_NPOINT = 1024
_RADIUS = 0.4
_NSAMPLE = 32
_TOPK = 512
_FG = 512


def _fps_kernel(pts_ref, idx_ref, *, npoint):
    R, M = pts_ref.shape[1], pts_ref.shape[2]
    x = pts_ref[0, :, :]
    y = pts_ref[1, :, :]
    z = pts_ref[2, :, :]
    lane = jax.lax.broadcasted_iota(jnp.int32, (R, M), 1)
    out_lane = jax.lax.broadcasted_iota(jnp.int32, (R, npoint), 1)
    out_row = jax.lax.broadcasted_iota(jnp.int32, (R, npoint), 0)
    out_lane2 = out_lane + out_row * npoint

    def body(i, carry):
        dists, far, idxs = carry
        hit = out_lane2 == (out_row * npoint + i)
        idxs = jnp.where(hit, jnp.broadcast_to(far, idxs.shape), idxs)
        sel = (lane == jnp.broadcast_to(far, lane.shape)).astype(jnp.float32)
        cx = jnp.sum(x * sel, axis=1, keepdims=True)
        cy = jnp.sum(y * sel, axis=1, keepdims=True)
        cz = jnp.sum(z * sel, axis=1, keepdims=True)
        d = (x - cx) ** 2 + (y - cy) ** 2 + (z - cz) ** 2
        dists = jnp.minimum(dists, d)
        mx = jnp.max(dists, axis=1, keepdims=True)
        far = jnp.min(jnp.where(dists == mx, lane, M),
                      axis=1, keepdims=True).astype(jnp.int32)
        return dists, far, idxs

    init = (
        jnp.full((R, M), 1e10, dtype=jnp.float32),
        jnp.zeros((R, 1), dtype=jnp.int32),
        jnp.zeros((R, npoint), dtype=jnp.int32),
    )
    _, _, idxs = jax.lax.fori_loop(0, npoint, body, init)
    idx_ref[...] = idxs


def _fps_call(pts, npoint):
    _, R, M = pts.shape
    return pl.pallas_call(
        functools.partial(_fps_kernel, npoint=npoint),
        in_specs=[pl.BlockSpec((3, R, M), lambda: (0, 0, 0))],
        out_specs=pl.BlockSpec((R, npoint), lambda: (0, 0)),
        out_shape=jax.ShapeDtypeStruct((R, npoint), jnp.int32),
    )(pts)


def _mlp_kernel(x_ref, w0_ref, b0_ref, w1_ref, b1_ref, w2_ref, b2_ref,
                o_ref, *, ts, ns):
    x = x_ref[...]
    y = jnp.maximum(jnp.dot(x, w0_ref[...],
                            preferred_element_type=jnp.float32) + b0_ref[...], 0.0)
    y = jnp.maximum(jnp.dot(y, w1_ref[...],
                            preferred_element_type=jnp.float32) + b1_ref[...], 0.0)
    y = jnp.maximum(jnp.dot(y, w2_ref[...],
                            preferred_element_type=jnp.float32) + b2_ref[...], 0.0)
    o_ref[...] = jnp.max(y.reshape(ts, ns, y.shape[-1]), axis=1)


def _mlp_call(grouped, w0t, b0, w1t, b1, w2t, b2):
    B, SN, Cin = grouped.shape
    ns = _NSAMPLE
    S = SN // ns
    Cout = w2t.shape[1]
    ts = 128
    kern = functools.partial(_mlp_kernel, ts=ts, ns=ns)
    rep = lambda b, s: (0, 0)
    out = pl.pallas_call(
        kern,
        grid=(B, S // ts),
        in_specs=[
            pl.BlockSpec((None, ts * ns, Cin), lambda b, s: (b, s, 0)),
            pl.BlockSpec(w0t.shape, rep),
            pl.BlockSpec(b0.shape, rep),
            pl.BlockSpec(w1t.shape, rep),
            pl.BlockSpec(b1.shape, rep),
            pl.BlockSpec(w2t.shape, rep),
            pl.BlockSpec(b2.shape, rep),
        ],
        out_specs=pl.BlockSpec((None, ts, Cout), lambda b, s: (b, s, 0)),
        out_shape=jax.ShapeDtypeStruct((B, S, Cout), jnp.float32),
    )(grouped, w0t, b0, w1t, b1, w2t, b2)
    return out


def kernel(xyz, features, fbs_w, fbs_b, conv_w0, bn_g0, bn_b0,
           conv_w1, bn_g1, bn_b1, conv_w2, bn_g2, bn_b2):
    B, N, _ = xyz.shape

    scores = jnp.einsum('oc,bcn->bon', fbs_w, features) + fbs_b[None, :, None]
    soft = jax.nn.softmax(scores, axis=1)
    margin = soft[:, 1, :] - soft[:, 0, :]
    _, top_idx = jax.lax.top_k(margin, _TOPK)
    mask = jnp.zeros((B, N), dtype=jnp.int32).at[
        jnp.arange(B)[:, None], top_idx].set(1)
    pos_idx = jnp.sort(top_idx, axis=1).astype(jnp.int32)
    neg_idx = jnp.argsort(mask, axis=1).astype(jnp.int32)[:, : N - _TOPK]

    gather_rows = jax.vmap(lambda a, i: a[i])
    pos_xyz = gather_rows(xyz, pos_idx)
    neg_xyz = gather_rows(xyz, neg_idx)
    M = neg_xyz.shape[1]
    pos_pad = jnp.concatenate(
        [pos_xyz, jnp.broadcast_to(pos_xyz[:, :1, :],
                                   (B, M - pos_xyz.shape[1], 3))], axis=1)
    allpts = jnp.concatenate([pos_pad, neg_xyz], axis=0)
    sel = _fps_call(jnp.transpose(allpts, (2, 0, 1)), _FG)
    sel_pos, sel_neg = sel[:B], sel[B:]
    new_pos = jnp.take_along_axis(pos_idx, sel_pos, axis=1)
    new_neg = jnp.take_along_axis(neg_idx, sel_neg, axis=1)
    indices = jnp.concatenate([new_pos, new_neg], axis=1)
    new_xyz = gather_rows(xyz, indices)

    def ball_one(xyz_b, new_xyz_b):
        sqr = jnp.sum((new_xyz_b[:, None, :] - xyz_b[None, :, :]) ** 2, axis=-1)
        gidx = jnp.broadcast_to(jnp.arange(N, dtype=jnp.int32), sqr.shape)
        gidx = jnp.where(sqr > _RADIUS ** 2, N, gidx)
        neg_top, _ = jax.lax.top_k(-gidx, _NSAMPLE)
        gidx = -neg_top
        first = gidx[:, :1]
        return jnp.where(gidx == N, first, gidx)

    gidx = jax.vmap(ball_one)(xyz, new_xyz)

    feat_t = jnp.transpose(features, (0, 2, 1))
    gx = gather_rows(xyz, gidx.reshape(B, -1)).reshape(B, _NPOINT, _NSAMPLE, 3)
    gx = gx - new_xyz[:, :, None, :]
    gf = gather_rows(feat_t, gidx.reshape(B, -1)).reshape(
        B, _NPOINT, _NSAMPLE, feat_t.shape[-1])
    grouped = jnp.concatenate([gx, gf], axis=-1)
    grouped = grouped.reshape(B, _NPOINT * _NSAMPLE, -1)

    w0t = (conv_w0 * bn_g0[:, None]).T
    w1t = (conv_w1 * bn_g1[:, None]).T
    w2t = (conv_w2 * bn_g2[:, None]).T
    out = _mlp_call(grouped, w0t, bn_b0[None, :], w1t, bn_b1[None, :],
                    w2t, bn_b2[None, :])
    new_features = jnp.transpose(out, (0, 2, 1))
    return new_xyz, new_features
